# baseline (speedup 1.0000x reference)
import functools

import jax
import jax.numpy as jnp
from jax.experimental import pallas as pl
from jax.experimental.pallas import tpu as pltpu

_ROWS_PER_BLOCK = 512


def _body(idx_ref, emb_ref, feat_ref, out_ref):
    row = emb_ref[pl.ds(idx_ref[0], 1), :]
    out_ref[...] = feat_ref[...] + row


def kernel(features, view_type_id, type_embedding):
    squeeze = False
    if features.ndim == 2:
        features = features[None, :, :]
        squeeze = True
    B, N, D = features.shape
    flat = features.reshape(B * N, D)
    rows = B * N
    rpb = _ROWS_PER_BLOCK if rows % _ROWS_PER_BLOCK == 0 else rows
    idx = jnp.asarray(view_type_id, jnp.int32).reshape(1)

    out = pl.pallas_call(
        _body,
        grid_spec=pltpu.PrefetchScalarGridSpec(
            num_scalar_prefetch=1,
            grid=(rows // rpb,),
            in_specs=[
                pl.BlockSpec(type_embedding.shape, lambda i, idx: (0, 0)),
                pl.BlockSpec((rpb, D), lambda i, idx: (i, 0)),
            ],
            out_specs=pl.BlockSpec((rpb, D), lambda i, idx: (i, 0)),
        ),
        out_shape=jax.ShapeDtypeStruct((rows, D), features.dtype),
    )(idx, type_embedding, flat)

    out = out.reshape(B, N, D)
    if squeeze:
        return out[0]
    return out
